# speedup vs baseline: 3.3349x; 3.3349x over previous
import jax
import jax.numpy as jnp
from jax import lax
from jax.experimental import pallas as pl
from jax.experimental.pallas import tpu as pltpu
from jax.experimental.pallas import tpu_sc as plsc

VOCAB = 100000
HIDDEN = 128
BATCH = 16384
HIST = 50

_INFO = plsc.get_sparse_core_info()
_NC = _INFO.num_cores
_NS = _INFO.num_subcores
_NW = _NC * _NS

_B = BATCH * HIST
_PER_W = _B // _NW
_CHUNK = 512
_NCHUNKS = _PER_W // _CHUNK


def _gather_body(idx_hbm, table_hbm, out_hbm, idx_v, rows_v, sem):
    wid = lax.axis_index("s") * _NC + lax.axis_index("c")
    base = wid * _PER_W

    def chunk(g, carry):
        row0 = base + g * _CHUNK
        pltpu.sync_copy(idx_hbm.at[pl.ds(row0, _CHUNK)], idx_v)
        pltpu.async_copy(table_hbm.at[idx_v], rows_v, sem).wait()
        pltpu.sync_copy(rows_v, out_hbm.at[pl.ds(row0, _CHUNK)])
        return carry

    lax.fori_loop(0, _NCHUNKS, chunk, 0)


@jax.jit
def kernel(inputs, weight):
    flat_idx = inputs.reshape(_B).astype(jnp.int32)
    mesh = plsc.VectorSubcoreMesh(core_axis_name="c", subcore_axis_name="s")
    run = pl.kernel(
        _gather_body,
        out_type=jax.ShapeDtypeStruct((_B, HIDDEN), jnp.float32),
        mesh=mesh,
        scratch_types=[
            pltpu.VMEM((_CHUNK,), jnp.int32),
            pltpu.VMEM((_CHUNK, HIDDEN), jnp.float32),
            pltpu.SemaphoreType.DMA,
        ],
    )
    out = run(flat_idx, weight)
    return out.reshape(BATCH, HIST, HIDDEN)


# speedup vs baseline: 3.4513x; 1.0349x over previous
import jax
import jax.numpy as jnp
from jax import lax
from jax.experimental import pallas as pl
from jax.experimental.pallas import tpu as pltpu
from jax.experimental.pallas import tpu_sc as plsc

VOCAB = 100000
HIDDEN = 128
BATCH = 16384
HIST = 50

_INFO = plsc.get_sparse_core_info()
_NC = _INFO.num_cores
_NS = _INFO.num_subcores
_NW = _NC * _NS

_B = BATCH * HIST
_PER_W = _B // _NW
_CHUNK = 400
_NCHUNKS = _PER_W // _CHUNK
_NPAIRS = _NCHUNKS // 2


def _gather_body(idx_hbm, table_hbm, out_hbm, idx_v, rows0, rows1,
                 gsem0, gsem1, ssem0, ssem1):
    wid = lax.axis_index("s") * _NC + lax.axis_index("c")
    base = wid * _PER_W

    pltpu.sync_copy(idx_hbm.at[pl.ds(base, _PER_W)], idx_v)

    def start_gather(i, rbuf, sem):
        pltpu.async_copy(table_hbm.at[idx_v.at[pl.ds(i * _CHUNK, _CHUNK)]],
                         rbuf, sem)

    def wait_gather(rbuf, sem):
        pltpu.make_async_copy(table_hbm.at[idx_v.at[pl.ds(0, _CHUNK)]],
                              rbuf, sem).wait()

    def start_store(i, rbuf, sem):
        pltpu.async_copy(rbuf, out_hbm.at[pl.ds(base + i * _CHUNK, _CHUNK)],
                         sem)

    def wait_store(rbuf, sem):
        pltpu.make_async_copy(rbuf, out_hbm.at[pl.ds(base, _CHUNK)],
                              sem).wait()

    start_gather(0, rows0, gsem0)

    def pair(g2, carry):
        i0 = 2 * g2
        wait_gather(rows0, gsem0)

        @pl.when(g2 > 0)
        def _():
            wait_store(rows1, ssem1)

        start_gather(i0 + 1, rows1, gsem1)
        start_store(i0, rows0, ssem0)

        wait_gather(rows1, gsem1)
        wait_store(rows0, ssem0)

        @pl.when(g2 < _NPAIRS - 1)
        def _():
            start_gather(i0 + 2, rows0, gsem0)

        start_store(i0 + 1, rows1, ssem1)
        return carry

    lax.fori_loop(0, _NPAIRS, pair, 0)
    wait_store(rows1, ssem1)


@jax.jit
def kernel(inputs, weight):
    flat_idx = inputs.reshape(_B).astype(jnp.int32)
    mesh = plsc.VectorSubcoreMesh(core_axis_name="c", subcore_axis_name="s")
    run = pl.kernel(
        _gather_body,
        out_type=jax.ShapeDtypeStruct((_B, HIDDEN), jnp.float32),
        mesh=mesh,
        scratch_types=[
            pltpu.VMEM((_PER_W,), jnp.int32),
            pltpu.VMEM((_CHUNK, HIDDEN), jnp.float32),
            pltpu.VMEM((_CHUNK, HIDDEN), jnp.float32),
            pltpu.SemaphoreType.DMA,
            pltpu.SemaphoreType.DMA,
            pltpu.SemaphoreType.DMA,
            pltpu.SemaphoreType.DMA,
        ],
    )
    out = run(flat_idx, weight)
    return out.reshape(BATCH, HIST, HIDDEN)


# speedup vs baseline: 6.2720x; 1.8173x over previous
import jax
import jax.numpy as jnp
from jax import lax
from jax.experimental import pallas as pl
from jax.experimental.pallas import tpu as pltpu
from jax.experimental.pallas import tpu_sc as plsc

VOCAB = 100000
HIDDEN = 128
BATCH = 16384
HIST = 50

_INFO = plsc.get_sparse_core_info()
_NC = _INFO.num_cores
_NS = _INFO.num_subcores
_NW = _NC * _NS

_B = BATCH * HIST
_PER_W = _B // _NW
_CB = 8
_CHUNK = _CB * HIST
_NCHUNKS = _PER_W // _CHUNK
_NPAIRS = _NCHUNKS // 2
_BATCH_PER_W = BATCH // _NW


def _gather_body(idx_hbm, table_hbm, out_hbm, idx_v, rows0, rows1,
                 gsem0, gsem1, ssem0, ssem1):
    wid = lax.axis_index("s") * _NC + lax.axis_index("c")
    base = wid * _PER_W
    bbase = wid * _BATCH_PER_W

    pltpu.sync_copy(idx_hbm.at[pl.ds(base, _PER_W)], idx_v)

    def start_gather(i, rbuf, sem):
        pltpu.async_copy(table_hbm.at[idx_v.at[pl.ds(i * _CHUNK, _CHUNK)]],
                         rbuf, sem)

    def wait_gather(rbuf, sem):
        pltpu.make_async_copy(table_hbm.at[idx_v.at[pl.ds(0, _CHUNK)]],
                              rbuf, sem).wait()

    def start_store(i, rbuf, sem):
        b0 = bbase + i * _CB
        for j in range(_CB):
            pltpu.async_copy(rbuf.at[pl.ds(j * HIST, HIST)],
                             out_hbm.at[b0 + j], sem)

    def wait_store(rbuf, sem):
        for j in range(_CB):
            pltpu.make_async_copy(rbuf.at[pl.ds(0, HIST)], out_hbm.at[0],
                                  sem).wait()

    start_gather(0, rows0, gsem0)

    def pair(g2, carry):
        i0 = 2 * g2
        wait_gather(rows0, gsem0)

        @pl.when(g2 > 0)
        def _():
            wait_store(rows1, ssem1)

        start_gather(i0 + 1, rows1, gsem1)
        start_store(i0, rows0, ssem0)

        wait_gather(rows1, gsem1)
        wait_store(rows0, ssem0)

        @pl.when(g2 < _NPAIRS - 1)
        def _():
            start_gather(i0 + 2, rows0, gsem0)

        start_store(i0 + 1, rows1, ssem1)
        return carry

    lax.fori_loop(0, _NPAIRS, pair, 0)
    wait_store(rows1, ssem1)


@jax.jit
def kernel(inputs, weight):
    flat_idx = inputs.reshape(_B).astype(jnp.int32)
    mesh = plsc.VectorSubcoreMesh(core_axis_name="c", subcore_axis_name="s")
    run = pl.kernel(
        _gather_body,
        out_type=jax.ShapeDtypeStruct((BATCH, HIST, HIDDEN), jnp.float32),
        mesh=mesh,
        scratch_types=[
            pltpu.VMEM((_PER_W,), jnp.int32),
            pltpu.VMEM((_CHUNK, HIDDEN), jnp.float32),
            pltpu.VMEM((_CHUNK, HIDDEN), jnp.float32),
            pltpu.SemaphoreType.DMA,
            pltpu.SemaphoreType.DMA,
            pltpu.SemaphoreType.DMA,
            pltpu.SemaphoreType.DMA,
        ],
        compiler_params=pltpu.CompilerParams(use_tc_tiling_on_sc=True),
    )
    return run(flat_idx, weight)


# speedup vs baseline: 11.9021x; 1.8977x over previous
import jax
import jax.numpy as jnp
from jax import lax
from jax.experimental import pallas as pl
from jax.experimental.pallas import tpu as pltpu
from jax.experimental.pallas import tpu_sc as plsc

VOCAB = 100000
HIDDEN = 128
BATCH = 16384
HIST = 50

_INFO = plsc.get_sparse_core_info()
_NC = _INFO.num_cores
_NS = _INFO.num_subcores
_NW = _NC * _NS

_B = BATCH * HIST
_PER_W = _B // _NW
_CHUNK = 400
_NCHUNKS = _PER_W // _CHUNK
_NPAIRS = _NCHUNKS // 2


def _gather_body(idx_hbm, table_hbm, out_hbm, idx_v, rows0, rows1,
                 gsem0, gsem1, ssem0, ssem1):
    wid = lax.axis_index("s") * _NC + lax.axis_index("c")
    base = wid * _PER_W

    pltpu.sync_copy(idx_hbm.at[pl.ds(base, _PER_W)], idx_v)

    def start_gather(i, rbuf, sem):
        pltpu.async_copy(table_hbm.at[idx_v.at[pl.ds(i * _CHUNK, _CHUNK)]],
                         rbuf, sem)

    def wait_gather(rbuf, sem):
        pltpu.make_async_copy(table_hbm.at[idx_v.at[pl.ds(0, _CHUNK)]],
                              rbuf, sem).wait()

    def start_store(i, rbuf, sem):
        pltpu.async_copy(rbuf, out_hbm.at[pl.ds(base + i * _CHUNK, _CHUNK)],
                         sem)

    def wait_store(rbuf, sem):
        pltpu.make_async_copy(rbuf, out_hbm.at[pl.ds(base, _CHUNK)],
                              sem).wait()

    start_gather(0, rows0, gsem0)

    def pair(g2, carry):
        i0 = 2 * g2
        wait_gather(rows0, gsem0)

        @pl.when(g2 > 0)
        def _():
            wait_store(rows1, ssem1)

        start_gather(i0 + 1, rows1, gsem1)
        start_store(i0, rows0, ssem0)

        wait_gather(rows1, gsem1)
        wait_store(rows0, ssem0)

        @pl.when(g2 < _NPAIRS - 1)
        def _():
            start_gather(i0 + 2, rows0, gsem0)

        start_store(i0 + 1, rows1, ssem1)
        return carry

    lax.fori_loop(0, _NPAIRS, pair, 0)
    wait_store(rows1, ssem1)


@jax.jit
def kernel(inputs, weight):
    flat_idx = inputs.astype(jnp.int32).T.reshape(_B)
    mesh = plsc.VectorSubcoreMesh(core_axis_name="c", subcore_axis_name="s")
    run = pl.kernel(
        _gather_body,
        out_type=jax.ShapeDtypeStruct((_B, HIDDEN), jnp.float32),
        mesh=mesh,
        scratch_types=[
            pltpu.VMEM((_PER_W,), jnp.int32),
            pltpu.VMEM((_CHUNK, HIDDEN), jnp.float32),
            pltpu.VMEM((_CHUNK, HIDDEN), jnp.float32),
            pltpu.SemaphoreType.DMA,
            pltpu.SemaphoreType.DMA,
            pltpu.SemaphoreType.DMA,
            pltpu.SemaphoreType.DMA,
        ],
    )
    out = run(flat_idx, weight)
    return out.reshape(HIST, BATCH, HIDDEN).transpose(1, 0, 2)


# speedup vs baseline: 12.0285x; 1.0106x over previous
import jax
import jax.numpy as jnp
from jax import lax
from jax.experimental import pallas as pl
from jax.experimental.pallas import tpu as pltpu
from jax.experimental.pallas import tpu_sc as plsc

VOCAB = 100000
HIDDEN = 128
BATCH = 16384
HIST = 50

_INFO = plsc.get_sparse_core_info()
_NC = _INFO.num_cores
_NS = _INFO.num_subcores
_NW = _NC * _NS

_B = BATCH * HIST
_PER_W = _B // _NW
_NBUF = 4
_CHUNK = 200
_NCHUNKS = _PER_W // _CHUNK
_NGROUPS = _NCHUNKS // _NBUF


def _gather_body(idx_hbm, table_hbm, out_hbm, idx_v, *bufs_and_sems):
    bufs = bufs_and_sems[:_NBUF]
    gsems = bufs_and_sems[_NBUF:2 * _NBUF]
    ssems = bufs_and_sems[2 * _NBUF:3 * _NBUF]

    wid = lax.axis_index("s") * _NC + lax.axis_index("c")
    base = wid * _PER_W

    pltpu.sync_copy(idx_hbm.at[pl.ds(base, _PER_W)], idx_v)

    def start_gather(i, b):
        pltpu.async_copy(table_hbm.at[idx_v.at[pl.ds(i * _CHUNK, _CHUNK)]],
                         bufs[b], gsems[b])

    def wait_gather(b):
        pltpu.make_async_copy(table_hbm.at[idx_v.at[pl.ds(0, _CHUNK)]],
                              bufs[b], gsems[b]).wait()

    def start_store(i, b):
        pltpu.async_copy(bufs[b], out_hbm.at[pl.ds(base + i * _CHUNK, _CHUNK)],
                         ssems[b])

    def wait_store(b):
        pltpu.make_async_copy(bufs[b], out_hbm.at[pl.ds(base, _CHUNK)],
                              ssems[b]).wait()

    for b in range(_NBUF - 1):
        start_gather(b, b)

    def group(g, carry):
        for b in range(_NBUF):
            i = g * _NBUF + b
            wait_gather(b)
            start_store(i, b)
            j = i + _NBUF - 1
            jb = (b + _NBUF - 1) % _NBUF

            def issue_next(j=j, jb=jb, b=b):
                if b == 0:
                    @pl.when(g > 0)
                    def _():
                        wait_store(jb)
                else:
                    wait_store(jb)
                start_gather(j, jb)

            @pl.when(j < _NCHUNKS)
            def _():
                issue_next()
        return carry

    lax.fori_loop(0, _NGROUPS, group, 0)
    for b in range(_NBUF):
        wait_store(b)


@jax.jit
def kernel(inputs, weight):
    flat_idx = inputs.astype(jnp.int32).T.reshape(_B)
    mesh = plsc.VectorSubcoreMesh(core_axis_name="c", subcore_axis_name="s")
    run = pl.kernel(
        _gather_body,
        out_type=jax.ShapeDtypeStruct((_B, HIDDEN), jnp.float32),
        mesh=mesh,
        scratch_types=(
            [pltpu.VMEM((_PER_W,), jnp.int32)]
            + [pltpu.VMEM((_CHUNK, HIDDEN), jnp.float32) for _ in range(_NBUF)]
            + [pltpu.SemaphoreType.DMA for _ in range(2 * _NBUF)]
        ),
    )
    out = run(flat_idx, weight)
    return out.reshape(HIST, BATCH, HIDDEN).transpose(1, 0, 2)
